# baseline (speedup 1.0000x reference)
import functools

import jax
import jax.numpy as jnp
from jax import lax
from jax.experimental import pallas as pl
from jax.experimental.pallas import tpu as pltpu
from jax.experimental.pallas import tpu_sc as plsc

_NC = 2
_NS = 16
_NW = _NC * _NS
_L = 16

_PT_THLD = 0.9
_LN2 = 0.6931471805599453
_UNROLL = 8

_MASK_WORDS = 4096

_LNP = (3.386965308216361e-08, 0.9999942724811793, -0.49983856183428216,
        0.3315486165205882, -0.23982616050327174, 0.16582275268978378,
        -0.09325203898561087, 0.03484971247846261, -0.0061514709617767945)


def _ln(x):
    bits = plsc.bitcast(x, jnp.int32)
    e = (bits >> 23) - 127
    m = plsc.bitcast((bits & 0x007FFFFF) | 0x3F800000, jnp.float32)
    t = m - 1.0
    p = _LNP[8]
    for c in _LNP[7::-1]:
        p = c + t * p
    return p + e.astype(jnp.float32) * _LN2


def _pack_mask(pt_padded, nwords, ngroups):
    def body(pt_ref, out_ref):
        acc = jnp.zeros((nwords,), jnp.int32)
        for j in range(ngroups):
            v = pt_ref[pl.ds(j * nwords, nwords)]
            acc = acc | ((v > _PT_THLD).astype(jnp.int32) << j)
        out_ref[...] = acc

    return pl.pallas_call(
        body, out_shape=jax.ShapeDtypeStruct((nwords,), jnp.int32)
    )(pt_padded)


def _make_sc_loss(n_edges, chunk):
    nchunks = n_edges // chunk
    npairs = nchunks // 2
    vecs = chunk // _L
    assert nchunks * chunk == n_edges and npairs * 2 == nchunks
    assert chunk % 128 == 0 and vecs % _UNROLL == 0
    mesh = plsc.VectorSubcoreMesh(core_axis_name="c", subcore_axis_name="s")

    @functools.partial(
        pl.kernel,
        out_type=jax.ShapeDtypeStruct((_NW, _L), jnp.float32),
        mesh=mesh,
        compiler_params=pltpu.CompilerParams(needs_layout_passes=False),
        scratch_types=[
            pltpu.VMEM((_MASK_WORDS,), jnp.int32),
            pltpu.VMEM((chunk,), jnp.float32),
            pltpu.VMEM((chunk,), jnp.float32),
            pltpu.VMEM((chunk,), jnp.int32),
            pltpu.VMEM((chunk,), jnp.int32),
            pltpu.VMEM((2, chunk), jnp.int32),
            pltpu.VMEM((2, chunk), jnp.int32),
            pltpu.VMEM((_L,), jnp.float32),
            pltpu.SemaphoreType.DMA,
            pltpu.SemaphoreType.DMA,
        ],
    )
    def sc_loss(w_hbm, y_hbm, ei_hbm, mask_hbm, out_hbm,
                mask_v, w0_v, w1_v, y0_v, y1_v, ei0_v, ei1_v,
                acc_v, sem0, sem1):
        wid = lax.axis_index("s") * _NC + lax.axis_index("c")
        w_v = (w0_v, w1_v)
        y_v = (y0_v, y1_v)
        ei_v = (ei0_v, ei1_v)
        sems = (sem0, sem1)
        cnt = (npairs - wid + _NW - 1) // _NW

        def copies(c, b):
            cbase = pl.multiple_of(c * chunk, 128)
            sl = pl.ds(cbase, chunk)
            return (
                pltpu.make_async_copy(w_hbm.at[sl], w_v[b], sems[b]),
                pltpu.make_async_copy(y_hbm.at[sl], y_v[b], sems[b]),
                pltpu.make_async_copy(ei_hbm.at[:, sl], ei_v[b], sems[b]),
            )

        def start(c, b):
            for cp in copies(c, b):
                cp.start()

        def wait(c, b):
            for cp in copies(c, b):
                cp.wait()

        def compute(b, acc):
            def vec_body(i, acc):
                sl = pl.ds(i, _L)
                idx = ei_v[b][0, sl]
                word = plsc.load_gather(mask_v, [idx & (_MASK_WORDS - 1)])
                bit = (word >> (idx >> 12)) & 1
                wv = w_v[b][sl]
                yv = y_v[b][sl]
                ym = (yv != 0) & (bit != 0)
                sel = jnp.where(ym, wv, 1.0 - wv)
                return acc - _ln(sel)

            return plsc.parallel_loop(0, chunk, _L, unroll=_UNROLL,
                                      carry=acc)(vec_body)

        start(2 * wid, 0)
        start(2 * wid + 1, 1)
        pltpu.sync_copy(mask_hbm, mask_v)

        def pair_body(k, acc):
            p = wid + k * _NW
            pn = wid + (lax.rem(k + 1, cnt)) * _NW
            wait(2 * p, 0)
            acc = compute(0, acc)
            start(2 * pn, 0)
            wait(2 * p + 1, 1)
            acc = compute(1, acc)
            start(2 * pn + 1, 1)
            return acc

        acc = lax.fori_loop(0, cnt, pair_body, jnp.zeros((_L,), jnp.float32))
        wait(2 * wid, 0)
        wait(2 * wid + 1, 1)
        acc_v[...] = acc
        pltpu.sync_copy(acc_v, out_hbm.at[wid])

    return sc_loss


def kernel(w, y, edge_index, pt):
    n_edges = w.shape[0]
    n_nodes = pt.shape[0]
    ngroups = (n_nodes + _MASK_WORDS - 1) // _MASK_WORDS
    pad = ngroups * _MASK_WORDS - n_nodes
    pt_padded = jnp.concatenate([pt, jnp.zeros((pad,), jnp.float32)])
    mask = _pack_mask(pt_padded, _MASK_WORDS, ngroups)
    sc_loss = _make_sc_loss(n_edges, chunk=12800)
    partials = sc_loss(w, y.astype(jnp.int32), edge_index.astype(jnp.int32),
                       mask)
    return jnp.sum(partials) / n_edges


# speedup vs baseline: 1.0439x; 1.0439x over previous
import functools

import jax
import jax.numpy as jnp
from jax import lax
from jax.experimental import pallas as pl
from jax.experimental.pallas import tpu as pltpu
from jax.experimental.pallas import tpu_sc as plsc

_NC = 2
_NS = 16
_NW = _NC * _NS
_L = 16

_PT_THLD = 0.9
_LN2 = 0.6931471805599453
_UNROLL = 8

_LNP = (3.386965308216361e-08, 0.9999942724811793, -0.49983856183428216,
        0.3315486165205882, -0.23982616050327174, 0.16582275268978378,
        -0.09325203898561087, 0.03484971247846261, -0.0061514709617767945)


def _ln(x):
    bits = plsc.bitcast(x, jnp.int32)
    e = (bits >> 23) - 127
    m = plsc.bitcast((bits & 0x007FFFFF) | 0x3F800000, jnp.float32)
    t = m - 1.0
    p = _LNP[8]
    for c in _LNP[7::-1]:
        p = c + t * p
    return p + e.astype(jnp.float32) * _LN2


def _make_sc_loss(n_edges, n_nodes, chunk):
    nchunks = n_edges // chunk
    npairs = nchunks // 2
    vecs = chunk // _L
    assert nchunks * chunk == n_edges and npairs * 2 == nchunks
    assert chunk % 128 == 0 and vecs % _UNROLL == 0
    mesh = plsc.VectorSubcoreMesh(core_axis_name="c", subcore_axis_name="s")

    @functools.partial(
        pl.kernel,
        out_type=jax.ShapeDtypeStruct((_NW, _L), jnp.float32),
        mesh=mesh,
        compiler_params=pltpu.CompilerParams(needs_layout_passes=False),
        scratch_types=[
            pltpu.VMEM((n_nodes,), jnp.float32),
            pltpu.VMEM((chunk,), jnp.float32),
            pltpu.VMEM((chunk,), jnp.float32),
            pltpu.VMEM((chunk,), jnp.int32),
            pltpu.VMEM((chunk,), jnp.int32),
            pltpu.VMEM((2, chunk), jnp.int32),
            pltpu.VMEM((2, chunk), jnp.int32),
            pltpu.VMEM((_L,), jnp.float32),
            pltpu.SemaphoreType.DMA,
            pltpu.SemaphoreType.DMA,
        ],
    )
    def sc_loss(w_hbm, y_hbm, ei_hbm, pt_hbm, out_hbm,
                pt_v, w0_v, w1_v, y0_v, y1_v, ei0_v, ei1_v,
                acc_v, sem0, sem1):
        wid = lax.axis_index("s") * _NC + lax.axis_index("c")
        w_v = (w0_v, w1_v)
        y_v = (y0_v, y1_v)
        ei_v = (ei0_v, ei1_v)
        sems = (sem0, sem1)
        cnt = (npairs - wid + _NW - 1) // _NW

        def copies(c, b):
            cbase = pl.multiple_of(c * chunk, 128)
            sl = pl.ds(cbase, chunk)
            return (
                pltpu.make_async_copy(w_hbm.at[sl], w_v[b], sems[b]),
                pltpu.make_async_copy(y_hbm.at[sl], y_v[b], sems[b]),
                pltpu.make_async_copy(ei_hbm.at[:, sl], ei_v[b], sems[b]),
            )

        def start(c, b):
            for cp in copies(c, b):
                cp.start()

        def wait(c, b):
            for cp in copies(c, b):
                cp.wait()

        def compute(b, acc):
            def vec_body(i, acc):
                sl = pl.ds(i, _L)
                idx = ei_v[b][0, sl]
                g = plsc.load_gather(pt_v, [idx])
                wv = w_v[b][sl]
                yv = y_v[b][sl]
                ym = (yv != 0) & (g > _PT_THLD)
                sel = jnp.where(ym, wv, 1.0 - wv)
                return acc - _ln(sel)

            return plsc.parallel_loop(0, chunk, _L, unroll=_UNROLL,
                                      carry=acc)(vec_body)

        start(2 * wid, 0)
        start(2 * wid + 1, 1)
        pltpu.sync_copy(pt_hbm, pt_v)

        def pair_body(k, acc):
            p = wid + k * _NW
            pn = wid + (lax.rem(k + 1, cnt)) * _NW
            wait(2 * p, 0)
            acc = compute(0, acc)
            start(2 * pn, 0)
            wait(2 * p + 1, 1)
            acc = compute(1, acc)
            start(2 * pn + 1, 1)
            return acc

        acc = lax.fori_loop(0, cnt, pair_body, jnp.zeros((_L,), jnp.float32))
        wait(2 * wid, 0)
        wait(2 * wid + 1, 1)
        acc_v[...] = acc
        pltpu.sync_copy(acc_v, out_hbm.at[wid])

    return sc_loss


def kernel(w, y, edge_index, pt):
    n_edges = w.shape[0]
    n_nodes = pt.shape[0]
    sc_loss = _make_sc_loss(n_edges, n_nodes, chunk=3200)
    partials = sc_loss(w, y.astype(jnp.int32), edge_index.astype(jnp.int32),
                       pt)
    return jnp.sum(partials) / n_edges
